# Initial kernel scaffold; baseline (speedup 1.0000x reference)
import jax
import jax.numpy as jnp
from jax.experimental import pallas as pl


def kernel(embeddings, edge_index, Wq, Wk, Wv):
    raise NotImplementedError("write your pallas kernel here")


# speedup vs baseline: 4.1426x; 4.1426x over previous
import functools

import jax
import jax.numpy as jnp
from jax import lax
from jax.experimental import pallas as pl
from jax.experimental.pallas import tpu as pltpu
from jax.experimental.pallas import tpu_sc as plsc

N_NODES = 10000
N_EDGES = 320000
EMBED = 128
LANES = 16
CHUNK = 128
N_CHUNKS = N_EDGES // CHUNK
N_WORKERS = 32
CHUNKS_PER_W = -(-N_CHUNKS // N_WORKERS)
N_PAD = 10240
ROWS_PER_SUB = N_PAD // 16
ZROWS = 128
SCALE = 1.0 / (EMBED ** 0.5)


def _proj_body(e_ref, wq_ref, wk_ref, wv_ref, q_ref, k_ref, v_ref):
    e = e_ref[...]
    dn = (((1,), (1,)), ((), ()))
    q_ref[...] = lax.dot_general(e, wq_ref[...], dn, preferred_element_type=jnp.float32)
    k_ref[...] = lax.dot_general(e, wk_ref[...], dn, preferred_element_type=jnp.float32)
    v_ref[...] = lax.dot_general(e, wv_ref[...], dn, preferred_element_type=jnp.float32)


def _project(e, wq, wk, wv):
    sds = jax.ShapeDtypeStruct((N_NODES, EMBED), jnp.float32)
    return pl.pallas_call(_proj_body, out_shape=[sds, sds, sds])(e, wq, wk, wv)


def _scores_body(q_hbm, k_hbm, src_hbm, dst_hbm, s_hbm,
                 src_v, dst_v, qrows, krows, s_v, sem_q, sem_k):
    c = lax.axis_index("c")
    s = lax.axis_index("s")
    wid = c * 16 + s

    def chunk_body(i, _):
        cid = wid * CHUNKS_PER_W + i

        @pl.when(cid < N_CHUNKS)
        def _():
            base = cid * CHUNK
            pltpu.sync_copy(src_hbm.at[pl.ds(base, CHUNK)], src_v)
            pltpu.sync_copy(dst_hbm.at[pl.ds(base, CHUNK)], dst_v)
            cq = pltpu.async_copy(q_hbm.at[src_v], qrows, sem_q)
            ck = pltpu.async_copy(k_hbm.at[dst_v], krows, sem_k)
            cq.wait()
            ck.wait()

            lane = lax.iota(jnp.int32, LANES)

            def group_body(g, _):
                grp = jnp.zeros((LANES,), jnp.float32)
                for l in range(LANES):
                    e = g * LANES + l
                    acc = qrows[e, pl.ds(0, LANES)] * krows[e, pl.ds(0, LANES)]
                    for j in range(1, EMBED // LANES):
                        acc = acc + (qrows[e, pl.ds(j * LANES, LANES)]
                                     * krows[e, pl.ds(j * LANES, LANES)])
                    for st in (1, 2, 4, 8):
                        acc = acc + acc.at[lane ^ st].get(
                            mode="promise_in_bounds")
                    grp = jnp.where(lane == l, acc, grp)
                s_v[pl.ds(g * LANES, LANES)] = grp * SCALE
                return 0

            lax.fori_loop(0, CHUNK // LANES, group_body, 0)
            pltpu.sync_copy(s_v, s_hbm.at[pl.ds(base, CHUNK)])

        return 0

    lax.fori_loop(0, CHUNKS_PER_W, chunk_body, 0)


def _edge_scores(q, k, src, dst):
    mesh = plsc.VectorSubcoreMesh(core_axis_name="c", subcore_axis_name="s")
    f = pl.kernel(
        _scores_body,
        out_type=jax.ShapeDtypeStruct((N_EDGES,), jnp.float32),
        mesh=mesh,
        scratch_types=[
            pltpu.VMEM((CHUNK,), jnp.int32),
            pltpu.VMEM((CHUNK,), jnp.int32),
            pltpu.VMEM((CHUNK, EMBED), jnp.float32),
            pltpu.VMEM((CHUNK, EMBED), jnp.float32),
            pltpu.VMEM((CHUNK,), jnp.float32),
            pltpu.SemaphoreType.DMA,
            pltpu.SemaphoreType.DMA,
        ],
    )
    return f(q, k, src, dst)


def _softmax_body(s_ref, w_ref):
    sc = s_ref[...]
    m = jnp.max(sc)
    e = jnp.exp(sc - m)
    w_ref[...] = e / jnp.sum(e)


def _softmax(scores):
    s2 = scores.reshape(N_CHUNKS, CHUNK)
    w2 = pl.pallas_call(
        _softmax_body,
        out_shape=jax.ShapeDtypeStruct((N_CHUNKS, CHUNK), jnp.float32),
    )(s2)
    return w2.reshape(N_EDGES)


def _scatter_body(v_hbm, src_hbm, dst_hbm, w_hbm, out_hbm,
                  src_v, dst_v, vrows, w_v, zbuf, accum, sem_v):
    c = lax.axis_index("c")
    s = lax.axis_index("s")
    wid = c * 16 + s

    def zrow(r, _):
        for j in range(EMBED // LANES):
            zbuf[r, pl.ds(j * LANES, LANES)] = jnp.zeros((LANES,), jnp.float32)
        return 0

    lax.fori_loop(0, ZROWS, zrow, 0)
    for kk in range(ROWS_PER_SUB // ZROWS):
        pltpu.sync_copy(zbuf, accum.at[pl.ds(s * ROWS_PER_SUB + kk * ZROWS, ZROWS)])
    plsc.subcore_barrier()

    def chunk_body(i, _):
        cid = wid * CHUNKS_PER_W + i

        @pl.when(cid < N_CHUNKS)
        def _():
            base = cid * CHUNK
            pltpu.sync_copy(src_hbm.at[pl.ds(base, CHUNK)], src_v)
            pltpu.sync_copy(dst_hbm.at[pl.ds(base, CHUNK)], dst_v)
            pltpu.sync_copy(w_hbm.at[pl.ds(base, CHUNK)], w_v)
            pltpu.async_copy(v_hbm.at[dst_v], vrows, sem_v).wait()

            def group_body(g, _):
                wvec = w_v[pl.ds(g * LANES, LANES)]
                for l in range(LANES):
                    e = g * LANES + l
                    we = wvec[l]
                    for j in range(EMBED // LANES):
                        sl = pl.ds(j * LANES, LANES)
                        vrows[e, sl] = vrows[e, sl] * we
                return 0

            lax.fori_loop(0, CHUNK // LANES, group_body, 0)
            pltpu.sync_copy(vrows, accum.at[src_v], add=True)

        return 0

    lax.fori_loop(0, CHUNKS_PER_W, chunk_body, 0)
    plsc.subcore_barrier()
    pltpu.sync_copy(accum.at[pl.ds(s * ROWS_PER_SUB, ROWS_PER_SUB)],
                    out_hbm.at[c, pl.ds(s * ROWS_PER_SUB, ROWS_PER_SUB)])


def _scatter(v, src, dst, w):
    mesh = plsc.VectorSubcoreMesh(core_axis_name="c", subcore_axis_name="s")
    f = pl.kernel(
        _scatter_body,
        out_type=jax.ShapeDtypeStruct((2, N_PAD, EMBED), jnp.float32),
        mesh=mesh,
        scratch_types=[
            pltpu.VMEM((CHUNK,), jnp.int32),
            pltpu.VMEM((CHUNK,), jnp.int32),
            pltpu.VMEM((CHUNK, EMBED), jnp.float32),
            pltpu.VMEM((CHUNK,), jnp.float32),
            pltpu.VMEM((ZROWS, EMBED), jnp.float32),
            pltpu.VMEM_SHARED((N_PAD, EMBED), jnp.float32),
            pltpu.SemaphoreType.DMA,
        ],
    )
    return f(v, src, dst, w)


def _combine_body(p_ref, o_ref):
    o_ref[...] = p_ref[0, :N_NODES] + p_ref[1, :N_NODES]


def _combine(parts):
    return pl.pallas_call(
        _combine_body,
        out_shape=jax.ShapeDtypeStruct((N_NODES, EMBED), jnp.float32),
    )(parts)


@jax.jit
def kernel(embeddings, edge_index, Wq, Wk, Wv):
    src = edge_index[0].astype(jnp.int32)
    dst = edge_index[1].astype(jnp.int32)
    q, k, v = _project(embeddings, Wq, Wk, Wv)
    scores = _edge_scores(q, k, src, dst)
    w = _softmax(scores)
    parts = _scatter(v, src, dst, w)
    return _combine(parts)
